# baseline (speedup 1.0000x reference)
import functools
import math

import jax
import jax.numpy as jnp
from jax import lax
from jax.experimental import pallas as pl
from jax.experimental.pallas import tpu as pltpu
from jax.experimental.pallas import tpu_sc as plsc

K = 2048
T = 512
D = 64
_C = 0.5 * D * math.log(2.0 * math.pi)
NSC = 16
P = K // NSC
NG = P // 16
ND = D // 16


def _sc_body(x_hbm, w_hbm, z2_hbm, u_hbm, out_hbm,
             xbuf, zbuf, wbuf, zidx, ubuf, cdfL, sigL, signew,
             stats_l, outm, outs,
             sigma_sh, cdf_sh, stats_sh,
             zsems, usems, wsem, ssem):
    sid = lax.axis_index("s")
    base = sid * P
    i16 = lax.iota(jnp.int32, 16)
    izero = jnp.zeros((16,), jnp.int32)
    fzero = jnp.zeros((16,), jnp.float32)

    pltpu.sync_copy(x_hbm, xbuf)
    pltpu.sync_copy(w_hbm.at[pl.ds(base, P)], wbuf)
    for g in range(NG):
        sig0 = base + g * 16 + i16
        signew[pl.ds(g * 16, 16)] = sig0
        zidx[pl.ds(g * 16, 16)] = sig0 * T
    pltpu.sync_copy(signew, sigma_sh.at[0, pl.ds(base, P)])
    pltpu.async_copy(z2_hbm.at[zidx], zbuf.at[0], zsems[0])
    pltpu.async_copy(u_hbm.at[0, pl.ds(base, P)], ubuf.at[0], usems[0])
    plsc.subcore_barrier()

    def one_step(t, buf):
        nbuf = 1 - buf
        for g in range(NG):
            zidx[pl.ds(g * 16, 16)] = zidx[pl.ds(g * 16, 16)] + 1

        @pl.when(t < T - 1)
        def _pref():
            pltpu.async_copy(z2_hbm.at[zidx], zbuf.at[nbuf], zsems[nbuf])
            pltpu.async_copy(u_hbm.at[t + 1, pl.ds(base, P)], ubuf.at[nbuf],
                             usems[nbuf])

        xv = [xbuf[pl.ds(t * D + 16 * i, 16)] for i in range(ND)]

        pltpu.make_async_copy(z2_hbm.at[zidx], zbuf.at[buf],
                              zsems[buf]).wait()

        bufv = izero + buf
        pidx = [g * 16 + i16 for g in range(NG)]

        def dotd(d, carry):
            A, B, Cc = carry
            dv = izero + d
            xd = plsc.load_gather(xbuf, [izero + (t * D + d)])
            A2, B2, C2 = [], [], []
            for g in range(NG):
                zv = plsc.load_gather(zbuf, [bufv, pidx[g], dv])
                wv = plsc.load_gather(wbuf, [pidx[g], dv])
                A2.append(A[g] + zv * (wv + xd))
                B2.append(B[g] + wv * wv)
                C2.append(Cc[g] + zv * zv)
            return (tuple(A2), tuple(B2), tuple(C2))

        z8 = (fzero,) * NG
        A, B, Cc = lax.fori_loop(0, D, dotd, (z8, z8, z8))
        sv = [A[g] - 0.5 * B[g] - 0.5 * Cc[g] for g in range(NG)]

        mv = sv[0]
        for g in range(1, NG):
            mv = jnp.maximum(mv, sv[g])
        m_loc = lax.reduce_max(mv, axes=(0,))
        carry_s = jnp.float32(0.0)
        for g in range(NG):
            ev = jnp.exp(sv[g] - m_loc)
            cs = plsc.cumsum(ev) + carry_s
            cdfL[pl.ds(g * 16, 16)] = cs
            carry_s = carry_s + lax.reduce_sum(ev, axes=(0,))

        row = jnp.where(i16 == 0, m_loc, jnp.where(i16 == 1, carry_s, 0.0))
        stats_l[0, :] = row
        pltpu.sync_copy(stats_l.at[0], stats_sh.at[buf, sid])
        pltpu.sync_copy(cdfL.at[pl.ds(0, P)], cdf_sh.at[buf, pl.ds(base, P)])
        plsc.subcore_barrier()

        pltpu.async_copy(sigma_sh.at[buf], sigL, ssem)
        pltpu.sync_copy(stats_sh.at[buf], stats_l)
        m_all = plsc.load_gather(stats_l, [i16, izero])
        s_all = plsc.load_gather(stats_l, [i16, izero + 1])
        m_glob = lax.reduce_max(m_all, axes=(0,))
        scale = jnp.exp(m_all - m_glob)
        s_sc = s_all * scale
        incl = plsc.cumsum(s_sc)
        pref = incl - s_sc
        S_tot = lax.reduce_sum(s_sc, axes=(0,))
        pltpu.sync_copy(cdf_sh.at[buf], cdfL)
        for j in range(NSC):
            sj = scale[j]
            pj = pref[j]
            for i in range(P // 16):
                off = j * P + i * 16
                cdfL[pl.ds(off, 16)] = cdfL[pl.ds(off, 16)] * sj + pj

        pltpu.make_async_copy(u_hbm.at[t, pl.ds(base, P)], ubuf.at[buf],
                              usems[buf]).wait()
        pltpu.make_async_copy(sigma_sh.at[buf], sigL, ssem).wait()
        for g in range(NG):
            tgt = ubuf[buf, pl.ds(g * 16, 16)] * S_tot
            pos = izero
            sz = K // 2
            while sz >= 1:
                cprobe = plsc.load_gather(cdfL, [pos + (sz - 1)])
                pos = jnp.where(cprobe < tgt, pos + sz, pos)
                sz //= 2
            signew[pl.ds(g * 16, 16)] = plsc.load_gather(sigL, [pos])
        pltpu.sync_copy(signew, sigma_sh.at[nbuf, pl.ds(base, P)])

        pltpu.async_copy(w_hbm.at[signew], wbuf, wsem)

        @pl.when(sid == 0)
        def _rec():
            cx = fzero
            for i in range(ND):
                cx = cx + xv[i] * xv[i]
            c_t = -0.5 * lax.reduce_sum(cx, axes=(0,)) - _C
            lane0 = i16 == 0
            tvec = izero + t
            plsc.store_scatter(outm, [tvec], fzero + (m_glob + c_t),
                               mask=lane0)
            plsc.store_scatter(outs, [tvec], fzero + S_tot, mask=lane0)

        pltpu.make_async_copy(w_hbm.at[signew], wbuf, wsem).wait()
        plsc.subcore_barrier()

    def pair(i, carry):
        one_step(2 * i, 0)
        one_step(2 * i + 1, 1)
        return carry

    lax.fori_loop(0, T // 2, pair, 0)

    @pl.when(sid == 0)
    def _out():
        pltpu.sync_copy(outm, out_hbm.at[0])
        pltpu.sync_copy(outs, out_hbm.at[1])


def _reduce_body(ms_ref, out_ref):
    ms = ms_ref[...]
    r = jnp.sum(ms[0:1, :] + jnp.log(ms[1:2, :]))
    out_ref[0, 0] = r - T * math.log(float(K))


@jax.jit
def kernel(x, w, z, u):
    z2 = z.reshape(K * T, D)
    mesh = plsc.VectorSubcoreMesh(core_axis_name="c", subcore_axis_name="s",
                                  num_cores=1)
    sc = pl.kernel(
        _sc_body,
        out_type=jax.ShapeDtypeStruct((2, T), jnp.float32),
        mesh=mesh,
        compiler_params=pltpu.CompilerParams(needs_layout_passes=False,
                                             use_tc_tiling_on_sc=False),
        scratch_types=[
            pltpu.VMEM((T * D,), jnp.float32),
            pltpu.VMEM((2, P, D), jnp.float32),
            pltpu.VMEM((P, D), jnp.float32),
            pltpu.VMEM((P,), jnp.int32),
            pltpu.VMEM((2, P), jnp.float32),
            pltpu.VMEM((K,), jnp.float32),
            pltpu.VMEM((K,), jnp.int32),
            pltpu.VMEM((P,), jnp.int32),
            pltpu.VMEM((NSC, 16), jnp.float32),
            pltpu.VMEM((T,), jnp.float32),
            pltpu.VMEM((T,), jnp.float32),
            pltpu.VMEM_SHARED((2, K), jnp.int32),
            pltpu.VMEM_SHARED((2, K), jnp.float32),
            pltpu.VMEM_SHARED((2, NSC, 16), jnp.float32),
            (pltpu.SemaphoreType.DMA, pltpu.SemaphoreType.DMA),
            (pltpu.SemaphoreType.DMA, pltpu.SemaphoreType.DMA),
            pltpu.SemaphoreType.DMA,
            pltpu.SemaphoreType.DMA,
        ],
    )
    ms = sc(x.reshape(T * D), w, z2, u)
    out = pl.pallas_call(
        _reduce_body,
        out_specs=pl.BlockSpec(memory_space=pltpu.SMEM),
        out_shape=jax.ShapeDtypeStruct((1, 1), jnp.float32),
    )(ms)
    return out[0, 0]


# speedup vs baseline: 3.8811x; 3.8811x over previous
import functools
import math

import jax
import jax.numpy as jnp
from jax import lax
from jax.experimental import pallas as pl
from jax.experimental.pallas import tpu as pltpu
from jax.experimental.pallas import tpu_sc as plsc

K = 2048
T = 512
D = 64
_C = 0.5 * D * math.log(2.0 * math.pi)
NSC = 16
P = K // NSC
NG = P // 16
ND = D // 16


def _sc_body(x_hbm, w_hbm, zt_hbm, u_hbm, out_hbm,
             xbuf, zbuf, wbuf, ubuf, cdfL, sigL, signew,
             stats_l, outm, outs,
             sigma_sh, cdf_sh, stats_sh, w_sh,
             zsems, usems, wsem, ssem):
    sid = lax.axis_index("s")
    base = sid * P
    i16 = lax.iota(jnp.int32, 16)
    izero = jnp.zeros((16,), jnp.int32)
    fzero = jnp.zeros((16,), jnp.float32)

    pltpu.sync_copy(x_hbm, xbuf)
    pltpu.sync_copy(w_hbm.at[pl.ds(base, P)], wbuf)
    pltpu.sync_copy(wbuf, w_sh.at[pl.ds(base, P)])
    for g in range(NG):
        sig0 = base + g * 16 + i16
        signew[pl.ds(g * 16, 16)] = sig0
    pltpu.sync_copy(signew, sigma_sh.at[0, pl.ds(base, P)])
    pltpu.async_copy(zt_hbm.at[0, pl.ds(base * D, P * D)], zbuf.at[0],
                     zsems[0])
    pltpu.async_copy(u_hbm.at[0, pl.ds(base, P)], ubuf.at[0], usems[0])
    plsc.subcore_barrier()

    def one_step(t, buf):
        nbuf = 1 - buf

        @pl.when(t < T - 1)
        def _pref():
            pltpu.async_copy(zt_hbm.at[t + 1, pl.ds(base * D, P * D)],
                             zbuf.at[nbuf], zsems[nbuf])
            pltpu.async_copy(u_hbm.at[t + 1, pl.ds(base, P)], ubuf.at[nbuf],
                             usems[nbuf])

        xv = [xbuf[pl.ds(t * D + 16 * i, 16)] for i in range(ND)]

        pltpu.make_async_copy(zt_hbm.at[t, pl.ds(base * D, P * D)],
                              zbuf.at[buf], zsems[buf]).wait()

        bufv = izero + buf
        pidx = [g * 16 + i16 for g in range(NG)]
        pidx64 = [(g * 16 + i16) * D for g in range(NG)]

        def dotd(d, carry):
            A, B, Cc = carry
            dv = izero + d
            xd = plsc.load_gather(xbuf, [izero + (t * D + d)])
            A2, B2, C2 = [], [], []
            for g in range(NG):
                zv = plsc.load_gather(zbuf, [bufv, pidx64[g] + dv])
                wv = plsc.load_gather(wbuf, [pidx[g], dv])
                A2.append(A[g] + zv * (wv + xd))
                B2.append(B[g] + wv * wv)
                C2.append(Cc[g] + zv * zv)
            return (tuple(A2), tuple(B2), tuple(C2))

        z8 = (fzero,) * NG
        A, B, Cc = lax.fori_loop(0, D, dotd, (z8, z8, z8))
        sv = [A[g] - 0.5 * B[g] - 0.5 * Cc[g] for g in range(NG)]

        mv = sv[0]
        for g in range(1, NG):
            mv = jnp.maximum(mv, sv[g])
        m_loc = lax.reduce_max(mv, axes=(0,))
        carry_s = jnp.float32(0.0)
        for g in range(NG):
            ev = jnp.exp(sv[g] - m_loc)
            cs = plsc.cumsum(ev) + carry_s
            cdfL[pl.ds(g * 16, 16)] = cs
            carry_s = carry_s + lax.reduce_sum(ev, axes=(0,))

        row = jnp.where(i16 == 0, m_loc, jnp.where(i16 == 1, carry_s, 0.0))
        stats_l[0, :] = row
        pltpu.sync_copy(stats_l.at[0], stats_sh.at[buf, sid])
        pltpu.sync_copy(cdfL.at[pl.ds(0, P)], cdf_sh.at[buf, pl.ds(base, P)])
        plsc.subcore_barrier()

        pltpu.async_copy(sigma_sh.at[buf], sigL, ssem)
        pltpu.sync_copy(stats_sh.at[buf], stats_l)
        m_all = plsc.load_gather(stats_l, [i16, izero])
        s_all = plsc.load_gather(stats_l, [i16, izero + 1])
        m_glob = lax.reduce_max(m_all, axes=(0,))
        scale = jnp.exp(m_all - m_glob)
        s_sc = s_all * scale
        incl = plsc.cumsum(s_sc)
        pref = incl - s_sc
        S_tot = lax.reduce_sum(s_sc, axes=(0,))
        pltpu.sync_copy(cdf_sh.at[buf], cdfL)
        for j in range(NSC):
            sj = scale[j]
            pj = pref[j]
            for i in range(P // 16):
                off = j * P + i * 16
                cdfL[pl.ds(off, 16)] = cdfL[pl.ds(off, 16)] * sj + pj

        pltpu.make_async_copy(u_hbm.at[t, pl.ds(base, P)], ubuf.at[buf],
                              usems[buf]).wait()
        pltpu.make_async_copy(sigma_sh.at[buf], sigL, ssem).wait()
        for g in range(NG):
            tgt = ubuf[buf, pl.ds(g * 16, 16)] * S_tot
            pos = izero
            sz = K // 2
            while sz >= 1:
                cprobe = plsc.load_gather(cdfL, [pos + (sz - 1)])
                pos = jnp.where(cprobe < tgt, pos + sz, pos)
                sz //= 2
            signew[pl.ds(g * 16, 16)] = plsc.load_gather(sigL, [pos])
        pltpu.sync_copy(signew, sigma_sh.at[nbuf, pl.ds(base, P)])

        pltpu.async_copy(w_sh.at[signew], wbuf, wsem)

        @pl.when(sid == 0)
        def _rec():
            cx = fzero
            for i in range(ND):
                cx = cx + xv[i] * xv[i]
            c_t = -0.5 * lax.reduce_sum(cx, axes=(0,)) - _C
            lane0 = i16 == 0
            tvec = izero + t
            plsc.store_scatter(outm, [tvec], fzero + (m_glob + c_t),
                               mask=lane0)
            plsc.store_scatter(outs, [tvec], fzero + S_tot, mask=lane0)

        pltpu.make_async_copy(w_sh.at[signew], wbuf, wsem).wait()
        plsc.subcore_barrier()

    def pair(i, carry):
        one_step(2 * i, 0)
        one_step(2 * i + 1, 1)
        return carry

    lax.fori_loop(0, T // 2, pair, 0)

    @pl.when(sid == 0)
    def _out():
        pltpu.sync_copy(outm, out_hbm.at[0])
        pltpu.sync_copy(outs, out_hbm.at[1])


def _reduce_body(ms_ref, out_ref):
    ms = ms_ref[...]
    r = jnp.sum(ms[0:1, :] + jnp.log(ms[1:2, :]))
    out_ref[0, 0] = r - T * math.log(float(K))


@jax.jit
def kernel(x, w, z, u):
    zt = jnp.transpose(z, (1, 0, 2)).reshape(T, K * D)
    mesh = plsc.VectorSubcoreMesh(core_axis_name="c", subcore_axis_name="s",
                                  num_cores=1)
    sc = pl.kernel(
        _sc_body,
        out_type=jax.ShapeDtypeStruct((2, T), jnp.float32),
        mesh=mesh,
        compiler_params=pltpu.CompilerParams(needs_layout_passes=False,
                                             use_tc_tiling_on_sc=False),
        scratch_types=[
            pltpu.VMEM((T * D,), jnp.float32),
            pltpu.VMEM((2, P * D), jnp.float32),
            pltpu.VMEM((P, D), jnp.float32),
            pltpu.VMEM((2, P), jnp.float32),
            pltpu.VMEM((K,), jnp.float32),
            pltpu.VMEM((K,), jnp.int32),
            pltpu.VMEM((P,), jnp.int32),
            pltpu.VMEM((NSC, 16), jnp.float32),
            pltpu.VMEM((T,), jnp.float32),
            pltpu.VMEM((T,), jnp.float32),
            pltpu.VMEM_SHARED((2, K), jnp.int32),
            pltpu.VMEM_SHARED((2, K), jnp.float32),
            pltpu.VMEM_SHARED((2, NSC, 16), jnp.float32),
            pltpu.VMEM_SHARED((K, D), jnp.float32),
            (pltpu.SemaphoreType.DMA, pltpu.SemaphoreType.DMA),
            (pltpu.SemaphoreType.DMA, pltpu.SemaphoreType.DMA),
            pltpu.SemaphoreType.DMA,
            pltpu.SemaphoreType.DMA,
        ],
    )
    ms = sc(x.reshape(T * D), w, zt, u)
    out = pl.pallas_call(
        _reduce_body,
        out_specs=pl.BlockSpec(memory_space=pltpu.SMEM),
        out_shape=jax.ShapeDtypeStruct((1, 1), jnp.float32),
    )(ms)
    return out[0, 0]
